# baseline (speedup 1.0000x reference)
import functools

import jax
import jax.numpy as jnp
from jax import lax
from jax.experimental import pallas as pl
from jax.experimental.pallas import tpu as pltpu


def _conv_pool_kernel(x_ref, w_ref, b_ref, pm_ref, feat_ref, poolx_ref):
    x = x_ref[0]
    feat = jnp.dot(w_ref[...], x.astype(jnp.bfloat16),
                   preferred_element_type=jnp.float32) + b_ref[...]
    feat_ref[0] = feat
    poolx_ref[0] = lax.dot_general(
        pm_ref[...], x, (((1,), (1,)), ((), ())),
        preferred_element_type=jnp.float32)


def _logit_loss_kernel(poolx_ref, w_ref, b_ref, emb_ref, lab_ref,
                       pred_ref, loss_ref, *, n_classes, pool_scale):
    pooled = lax.dot_general(
        poolx_ref[...] * pool_scale, w_ref[...],
        (((1,), (1,)), ((), ())),
        preferred_element_type=jnp.float32) + b_ref[...]
    logits = jnp.dot(pooled, emb_ref[...],
                     preferred_element_type=jnp.float32)
    col = lax.broadcasted_iota(jnp.int32, logits.shape, 1)
    logits = jnp.where(col < n_classes, logits, -1e30)
    m = jnp.max(logits, axis=1, keepdims=True)
    pred_ref[...] = jnp.min(
        jnp.where(logits == m, col, logits.shape[1]),
        axis=1, keepdims=True).astype(jnp.int32)
    lse = m + jnp.log(jnp.sum(jnp.exp(logits - m), axis=1, keepdims=True))
    correct = jnp.sum(jnp.where(col == lab_ref[...], logits, 0.0),
                      axis=1, keepdims=True)
    loss_ref[...] = jnp.sum(lse - correct).reshape(1, 1, 1)


def kernel(latent, labels, emb_dict, conv_w, conv_b):
    B, Cin, H, W = latent.shape
    Cemb = conv_w.shape[0]
    P = 2
    PP = P * P
    HW = H * W
    hb, wb = H // P, W // P
    K = int(emb_dict.shape[0])
    K_pad = max(128, ((K + 127) // 128) * 128)
    R = B * PP

    x3 = latent.reshape(B, Cin, HW)
    w_mat = conv_w.reshape(Cemb, Cin)
    w_bf = w_mat.astype(jnp.bfloat16)
    b_col = conv_b.reshape(Cemb, 1).astype(jnp.float32)
    b_row = conv_b.reshape(1, Cemb).astype(jnp.float32)

    hi = jnp.arange(H) // hb
    wi = jnp.arange(W) // wb
    pid = (hi[:, None] * P + wi[None, :]).reshape(HW)
    pm = jax.nn.one_hot(pid, PP, dtype=jnp.float32).T

    emb_t = jnp.zeros((Cemb, K_pad), jnp.float32).at[:, :K].set(
        emb_dict.astype(jnp.float32).T)
    labels_rep = jnp.repeat(labels.astype(jnp.int32), PP).reshape(R, 1)

    itemsize = 4
    feat, poolx = pl.pallas_call(
        _conv_pool_kernel,
        grid=(B,),
        in_specs=[
            pl.BlockSpec((1, Cin, HW), lambda b: (b, 0, 0)),
            pl.BlockSpec((Cemb, Cin), lambda b: (0, 0)),
            pl.BlockSpec((Cemb, 1), lambda b: (0, 0)),
            pl.BlockSpec((PP, HW), lambda b: (0, 0)),
        ],
        out_specs=[
            pl.BlockSpec((1, Cemb, HW), lambda b: (b, 0, 0)),
            pl.BlockSpec((1, PP, Cin), lambda b: (b, 0, 0)),
        ],
        out_shape=[
            jax.ShapeDtypeStruct((B, Cemb, HW), jnp.float32),
            jax.ShapeDtypeStruct((B, PP, Cin), jnp.float32),
        ],
        compiler_params=pltpu.CompilerParams(
            dimension_semantics=("parallel",)),
        cost_estimate=pl.CostEstimate(
            flops=int(2 * B * HW * Cin * Cemb + 2 * B * HW * Cin * PP),
            transcendentals=0,
            bytes_accessed=int(B * HW * (Cin + Cemb) * itemsize
                               + (Cemb * Cin + Cemb) * itemsize
                               + B * PP * Cin * itemsize)),
    )(x3, w_bf, b_col, pm)

    RH = R // 2
    pred2, loss2 = pl.pallas_call(
        functools.partial(_logit_loss_kernel, n_classes=K,
                          pool_scale=1.0 / float(hb * wb)),
        grid=(2,),
        in_specs=[
            pl.BlockSpec((RH, Cin), lambda i: (i, 0)),
            pl.BlockSpec((Cemb, Cin), lambda i: (0, 0)),
            pl.BlockSpec((1, Cemb), lambda i: (0, 0)),
            pl.BlockSpec((Cemb, K_pad), lambda i: (0, 0)),
            pl.BlockSpec((RH, 1), lambda i: (i, 0)),
        ],
        out_specs=[
            pl.BlockSpec((RH, 1), lambda i: (i, 0)),
            pl.BlockSpec((1, 1, 1), lambda i: (i, 0, 0)),
        ],
        out_shape=[
            jax.ShapeDtypeStruct((R, 1), jnp.int32),
            jax.ShapeDtypeStruct((2, 1, 1), jnp.float32),
        ],
        compiler_params=pltpu.CompilerParams(
            dimension_semantics=("parallel",)),
        cost_estimate=pl.CostEstimate(
            flops=int(2 * R * Cin * Cemb + 2 * R * Cemb * K_pad),
            transcendentals=int(R * K_pad),
            bytes_accessed=int((R * Cin + Cemb * Cin + Cemb * K_pad) * itemsize
                               + R * 8)),
    )(poolx.reshape(R, Cin), w_mat.astype(jnp.float32), b_row, emb_t,
      labels_rep)

    x_out = feat.reshape(B, Cemb, H, W)
    pred = pred2.reshape(R)
    loss = jnp.sum(loss2) / float(R)
    return x_out, pred, loss, {'dict_loss': loss}
